# baseline (speedup 1.0000x reference)
import functools

import jax
import jax.numpy as jnp
from jax.experimental import pallas as pl
from jax.experimental.pallas import tpu as pltpu

_K = 5
_P = 2


def _block_body(x_ref, s_ref, b_ref, w_ref, o_ref, ypad_ref, *, NB, D, H, WC):
    rows = NB * D * H

    y = jnp.maximum(x_ref[...] * s_ref[...] + b_ref[...], 0.0)

    ypad_ref[...] = jnp.zeros_like(ypad_ref)
    ypad_ref[:, _P:_P + D, _P:_P + H, :] = (
        y.reshape(NB, D, H, WC).astype(ypad_ref.dtype))

    parts = []
    for kd in range(_K):
        for dh in range(_K):
            parts.append(ypad_ref[:, kd:kd + D, dh:dh + H, :])
    lhs = jnp.concatenate(parts, axis=-1).reshape(rows, _K * _K * WC)

    o_ref[...] = jnp.dot(lhs, w_ref[...], preferred_element_type=jnp.float32)


@jax.jit
def kernel(x, scale_t, bias_t, w_mat):
    N, D, H, W, Cin = x.shape
    WC = W * Cin
    L_out = w_mat.shape[1]
    Cout = L_out // (H * W)
    WCo = W * Cout

    x2 = x.reshape(N * D * H, WC)

    s = scale_t[:, :WC]
    b = bias_t[:, :WC]

    w_c = w_mat.reshape(_K, H, WC, L_out)[:, :_K, :, _P * WCo:(_P + 1) * WCo]
    w_c = w_c.reshape(_K * _K * WC, WCo)

    CORES = 2
    NB = N // CORES
    rows = NB * D * H

    body = functools.partial(_block_body, NB=NB, D=D, H=H, WC=WC)

    out = pl.pallas_call(
        body,
        out_shape=jax.ShapeDtypeStruct((N * D * H, WCo), jnp.float32),
        grid_spec=pltpu.PrefetchScalarGridSpec(
            num_scalar_prefetch=0,
            grid=(CORES,),
            in_specs=[
                pl.BlockSpec((rows, WC), lambda i: (i, 0)),
                pl.BlockSpec((1, WC), lambda i: (0, 0)),
                pl.BlockSpec((1, WC), lambda i: (0, 0)),
                pl.BlockSpec((_K * _K * WC, WCo), lambda i: (0, 0)),
            ],
            out_specs=pl.BlockSpec((rows, WCo), lambda i: (i, 0)),
            scratch_shapes=[
                pltpu.VMEM((NB, D + 2 * _P, H + 2 * _P, WC), jnp.bfloat16),
            ],
        ),
        compiler_params=pltpu.CompilerParams(
            dimension_semantics=("parallel",),
            vmem_limit_bytes=64 * 1024 * 1024),
    )(x2, s, b, w_c)

    return out.reshape(N, D, H, W, Cout)


# speedup vs baseline: 1.3862x; 1.3862x over previous
import functools

import jax
import jax.numpy as jnp
from jax.experimental import pallas as pl
from jax.experimental.pallas import tpu as pltpu

_K = 5
_P = 2


def _block_body(x_ref, s_ref, b_ref, *rest, NB, D, H, WC):
    w_refs = rest[:_K * _K]
    o_ref, wcat_ref, ypad_ref = rest[_K * _K:]
    rows = NB * D * H

    for t in range(_K * _K):
        wcat_ref[t * WC:(t + 1) * WC, :] = w_refs[t][...]

    y = jnp.maximum(x_ref[...] * s_ref[...] + b_ref[...], 0.0)

    ypad_ref[...] = jnp.zeros_like(ypad_ref)
    ypad_ref[:, _P:_P + D, _P:_P + H, :] = (
        y.reshape(NB, D, H, WC).astype(ypad_ref.dtype))

    parts = []
    for kd in range(_K):
        for dh in range(_K):
            parts.append(ypad_ref[:, kd:kd + D, dh:dh + H, :])
    lhs = jnp.concatenate(parts, axis=-1).reshape(rows, _K * _K * WC)

    o_ref[...] = jnp.dot(lhs, wcat_ref[...],
                         preferred_element_type=jnp.float32)


@jax.jit
def kernel(x, scale_t, bias_t, w_mat):
    N, D, H, W, Cin = x.shape
    WC = W * Cin
    L_out = w_mat.shape[1]
    Cout = L_out // (H * W)
    WCo = W * Cout

    x2 = x.reshape(N * D * H, WC)

    CORES = 2
    NB = N // CORES
    rows = NB * D * H

    body = functools.partial(_block_body, NB=NB, D=D, H=H, WC=WC)

    w_specs = [
        pl.BlockSpec((WC, WCo), functools.partial(
            lambda kd, dh, i: (kd * H + dh, _P), kd, dh))
        for kd in range(_K) for dh in range(_K)
    ]

    out = pl.pallas_call(
        body,
        out_shape=jax.ShapeDtypeStruct((N * D * H, WCo), jnp.float32),
        grid_spec=pltpu.PrefetchScalarGridSpec(
            num_scalar_prefetch=0,
            grid=(CORES,),
            in_specs=[
                pl.BlockSpec((rows, WC), lambda i: (i, 0)),
                pl.BlockSpec((1, WC), lambda i: (0, 0)),
                pl.BlockSpec((1, WC), lambda i: (0, 0)),
            ] + w_specs,
            out_specs=pl.BlockSpec((rows, WCo), lambda i: (i, 0)),
            scratch_shapes=[
                pltpu.VMEM((_K * _K * WC, WCo), jnp.bfloat16),
                pltpu.VMEM((NB, D + 2 * _P, H + 2 * _P, WC), jnp.bfloat16),
            ],
        ),
        compiler_params=pltpu.CompilerParams(
            dimension_semantics=("parallel",),
            vmem_limit_bytes=64 * 1024 * 1024),
    )(x2, scale_t, bias_t, *([w_mat] * (_K * _K)))

    return out.reshape(N, D, H, W, Cout)
